# baseline (speedup 1.0000x reference)
import functools

import jax
import jax.numpy as jnp
from jax.experimental import pallas as pl
from jax.experimental.pallas import tpu as pltpu

N_EXPERTS = 8
TOP_K = 2
D_MODEL = 1024
D_FF = 2048
BF = 512
NF = D_FF // BF


def _moe_kernel(x_ref, wr_ref, wg_ref, wu_ref, wd_ref,
                out_ref, aux_ref, comb_ref):
    e = pl.program_id(0)
    f = pl.program_id(1)
    xf = x_ref[...]

    @pl.when((e == 0) & (f == 0))
    def _routing():
        logits = jax.lax.dot_general(
            xf, wr_ref[...], (((1,), (1,)), ((), ())),
            preferred_element_type=jnp.float32)
        m = jnp.max(logits, axis=-1, keepdims=True)
        ex = jnp.exp(logits - m)
        scores = ex / jnp.sum(ex, axis=-1, keepdims=True)
        iota = jax.lax.broadcasted_iota(jnp.int32, scores.shape, 1)
        m1 = jnp.max(scores, axis=-1, keepdims=True)
        i1 = jnp.min(jnp.where(scores == m1, iota, N_EXPERTS),
                     axis=-1, keepdims=True)
        sel1 = iota == i1
        masked = jnp.where(sel1, -jnp.inf, scores)
        m2 = jnp.max(masked, axis=-1, keepdims=True)
        i2 = jnp.min(jnp.where(masked == m2, iota, N_EXPERTS),
                     axis=-1, keepdims=True)
        sel = sel1 | (iota == i2)
        comb_ref[...] = jnp.where(sel, scores, 0.0) / (m1 + m2)
        probs = jnp.mean(scores, axis=0, keepdims=True)
        fracs = jnp.mean(sel.astype(jnp.float32), axis=0, keepdims=True)
        aux_ref[...] = N_EXPERTS * jnp.sum(probs * fracs, keepdims=True)
        out_ref[...] = jnp.zeros_like(out_ref)

    x16 = xf.astype(jnp.bfloat16)
    g = jax.lax.dot_general(x16, wg_ref[0].astype(jnp.bfloat16),
                            (((1,), (1,)), ((), ())),
                            preferred_element_type=jnp.float32)
    u = jax.lax.dot_general(x16, wu_ref[0].astype(jnp.bfloat16),
                            (((1,), (1,)), ((), ())),
                            preferred_element_type=jnp.float32)
    h = (g * jax.lax.logistic(g)) * u
    iota = jax.lax.broadcasted_iota(jnp.int32, (comb_ref.shape[0], N_EXPERTS), 1)
    c_e = jnp.sum(jnp.where(iota == e, comb_ref[...], 0.0),
                  axis=-1, keepdims=True)
    hs = (h * c_e).astype(jnp.bfloat16)
    y = jax.lax.dot_general(hs, wd_ref[0].astype(jnp.bfloat16),
                            (((1,), (1,)), ((), ())),
                            preferred_element_type=jnp.float32)
    out_ref[...] += y


@functools.partial(jax.jit, static_argnums=())
def kernel(x, W_router, W_gate, W_up, W_down):
    Bx, Tx, D = x.shape
    xf = x.reshape(-1, D)
    n = xf.shape[0]
    out, aux = pl.pallas_call(
        _moe_kernel,
        grid=(N_EXPERTS, NF),
        in_specs=[
            pl.BlockSpec((n, D), lambda e, f: (0, 0)),
            pl.BlockSpec((N_EXPERTS, D), lambda e, f: (0, 0)),
            pl.BlockSpec((1, BF, D), lambda e, f: (e, f, 0)),
            pl.BlockSpec((1, BF, D), lambda e, f: (e, f, 0)),
            pl.BlockSpec((1, D, BF), lambda e, f: (e, 0, f)),
        ],
        out_specs=[
            pl.BlockSpec((n, D), lambda e, f: (0, 0)),
            pl.BlockSpec((1, 1), lambda e, f: (0, 0)),
        ],
        out_shape=[
            jax.ShapeDtypeStruct((n, D), jnp.float32),
            jax.ShapeDtypeStruct((1, 1), jnp.float32),
        ],
        scratch_shapes=[pltpu.VMEM((n, N_EXPERTS), jnp.float32)],
        compiler_params=pltpu.CompilerParams(
            dimension_semantics=("arbitrary", "arbitrary"),
        ),
    )(xf, W_router, W_gate, W_up, W_down)
    return out.reshape(Bx, Tx, D), aux[0, 0]


# speedup vs baseline: 1.0896x; 1.0896x over previous
import functools

import jax
import jax.numpy as jnp
from jax.experimental import pallas as pl
from jax.experimental.pallas import tpu as pltpu

N_EXPERTS = 8
D_MODEL = 1024
D_FF = 2048
HF = D_FF // 2
HD = D_MODEL // 2


def _moe_kernel(x_ref, wr_ref, wg_hbm, wu_hbm, wd_hbm,
                out_ref, aux_ref, bufA, bufD, sems):
    xf = x_ref[...]
    x16 = xf.astype(jnp.bfloat16)

    def issue(e, j):
        if j == 0:
            cp = pltpu.make_async_copy(
                wg_hbm.at[e, pl.ds(0, HF), :], bufA.at[0], sems.at[0])
        elif j == 1:
            cp = pltpu.make_async_copy(
                wu_hbm.at[e, pl.ds(0, HF), :], bufA.at[1], sems.at[1])
        elif j == 2:
            cp = pltpu.make_async_copy(
                wg_hbm.at[e, pl.ds(HF, HF), :], bufA.at[2], sems.at[2])
        elif j == 3:
            cp = pltpu.make_async_copy(
                wu_hbm.at[e, pl.ds(HF, HF), :], bufA.at[3], sems.at[3])
        elif j == 4:
            cp = pltpu.make_async_copy(
                wd_hbm.at[e, pl.ds(0, HD), :], bufD.at[0], sems.at[4])
        else:
            cp = pltpu.make_async_copy(
                wd_hbm.at[e, pl.ds(HD, HD), :], bufD.at[1], sems.at[5])
        cp.start()

    def wait(e, j):
        if j in (0, 1, 2, 3):
            pltpu.make_async_copy(
                wg_hbm.at[e, pl.ds(0, HF), :], bufA.at[j], sems.at[j]).wait()
        else:
            pltpu.make_async_copy(
                wd_hbm.at[e, pl.ds(0, HD), :], bufD.at[j - 4], sems.at[j]).wait()

    for j in range(6):
        issue(0, j)

    logits = jax.lax.dot_general(
        xf, wr_ref[...], (((1,), (1,)), ((), ())),
        preferred_element_type=jnp.float32)
    m = jnp.max(logits, axis=-1, keepdims=True)
    ex = jnp.exp(logits - m)
    scores = ex / jnp.sum(ex, axis=-1, keepdims=True)
    iota = jax.lax.broadcasted_iota(jnp.int32, scores.shape, 1)
    m1 = jnp.max(scores, axis=-1, keepdims=True)
    i1 = jnp.min(jnp.where(scores == m1, iota, N_EXPERTS),
                 axis=-1, keepdims=True)
    sel1 = iota == i1
    masked = jnp.where(sel1, -jnp.inf, scores)
    m2 = jnp.max(masked, axis=-1, keepdims=True)
    i2 = jnp.min(jnp.where(masked == m2, iota, N_EXPERTS),
                 axis=-1, keepdims=True)
    sel = sel1 | (iota == i2)
    comb = jnp.where(sel, scores, 0.0) / (m1 + m2)
    probs = jnp.mean(scores, axis=0, keepdims=True)
    fracs = jnp.mean(sel.astype(jnp.float32), axis=0, keepdims=True)
    aux_ref[...] = N_EXPERTS * jnp.sum(probs * fracs, keepdims=True)
    out_ref[...] = jnp.zeros_like(out_ref)

    def body(e, carry):
        c_e = jnp.sum(jnp.where(iota == e, comb, 0.0),
                      axis=-1, keepdims=True)

        def step(j, compute):
            wait(e, j)
            res = compute()
            @pl.when(e + 1 < N_EXPERTS)
            def _():
                issue(e + 1, j)
            return res

        g0 = step(0, lambda: jax.lax.dot_general(
            x16, bufA[0].astype(jnp.bfloat16), (((1,), (1,)), ((), ())),
            preferred_element_type=jnp.float32))
        u0 = step(1, lambda: jax.lax.dot_general(
            x16, bufA[1].astype(jnp.bfloat16), (((1,), (1,)), ((), ())),
            preferred_element_type=jnp.float32))
        hs0 = ((g0 * jax.lax.logistic(g0)) * u0 * c_e).astype(jnp.bfloat16)
        g1 = step(2, lambda: jax.lax.dot_general(
            x16, bufA[2].astype(jnp.bfloat16), (((1,), (1,)), ((), ())),
            preferred_element_type=jnp.float32))
        u1 = step(3, lambda: jax.lax.dot_general(
            x16, bufA[3].astype(jnp.bfloat16), (((1,), (1,)), ((), ())),
            preferred_element_type=jnp.float32))
        hs1 = ((g1 * jax.lax.logistic(g1)) * u1 * c_e).astype(jnp.bfloat16)
        hs = jnp.concatenate([hs0, hs1], axis=1)
        y0 = step(4, lambda: jax.lax.dot_general(
            hs, bufD[0].astype(jnp.bfloat16), (((1,), (1,)), ((), ())),
            preferred_element_type=jnp.float32))
        out_ref[:, 0:HD] += y0
        y1 = step(5, lambda: jax.lax.dot_general(
            hs, bufD[1].astype(jnp.bfloat16), (((1,), (1,)), ((), ())),
            preferred_element_type=jnp.float32))
        out_ref[:, HD:D_MODEL] += y1
        return carry

    jax.lax.fori_loop(0, N_EXPERTS, body, 0)


@functools.partial(jax.jit, static_argnums=())
def kernel(x, W_router, W_gate, W_up, W_down):
    Bx, Tx, D = x.shape
    xf = x.reshape(-1, D)
    n = xf.shape[0]
    out, aux = pl.pallas_call(
        _moe_kernel,
        in_specs=[
            pl.BlockSpec(memory_space=pltpu.MemorySpace.VMEM),
            pl.BlockSpec(memory_space=pltpu.MemorySpace.VMEM),
            pl.BlockSpec(memory_space=pltpu.MemorySpace.HBM),
            pl.BlockSpec(memory_space=pltpu.MemorySpace.HBM),
            pl.BlockSpec(memory_space=pltpu.MemorySpace.HBM),
        ],
        out_specs=[
            pl.BlockSpec(memory_space=pltpu.MemorySpace.VMEM),
            pl.BlockSpec(memory_space=pltpu.MemorySpace.VMEM),
        ],
        out_shape=[
            jax.ShapeDtypeStruct((n, D), jnp.float32),
            jax.ShapeDtypeStruct((1, 1), jnp.float32),
        ],
        scratch_shapes=[
            pltpu.VMEM((4, HF, D_MODEL), jnp.float32),
            pltpu.VMEM((2, HD, D_FF), jnp.float32),
            pltpu.SemaphoreType.DMA((6,)),
        ],
    )(xf, W_router, W_gate, W_up, W_down)
    return out.reshape(Bx, Tx, D), aux[0, 0]


# speedup vs baseline: 1.1714x; 1.0751x over previous
import functools

import jax
import jax.numpy as jnp
from jax.experimental import pallas as pl
from jax.experimental.pallas import tpu as pltpu

N_EXPERTS = 8
D_MODEL = 1024
D_FF = 2048
HF = D_FF // 2
HD = D_MODEL // 2


def _moe_kernel(x_ref, wr_ref, wg_hbm, wu_hbm, wd_hbm,
                out_ref, aux_ref, bufA, bufD, acc_ref, sems):
    xf = x_ref[:, 0, :]
    x16 = xf.astype(jnp.bfloat16)

    def issue(e, j):
        if j == 0:
            cp = pltpu.make_async_copy(
                wg_hbm.at[e, pl.ds(0, HF), :], bufA.at[0], sems.at[0])
        elif j == 1:
            cp = pltpu.make_async_copy(
                wu_hbm.at[e, pl.ds(0, HF), :], bufA.at[1], sems.at[1])
        elif j == 2:
            cp = pltpu.make_async_copy(
                wg_hbm.at[e, pl.ds(HF, HF), :], bufA.at[2], sems.at[2])
        elif j == 3:
            cp = pltpu.make_async_copy(
                wu_hbm.at[e, pl.ds(HF, HF), :], bufA.at[3], sems.at[3])
        elif j == 4:
            cp = pltpu.make_async_copy(
                wd_hbm.at[e, pl.ds(0, HD), :], bufD.at[0], sems.at[4])
        else:
            cp = pltpu.make_async_copy(
                wd_hbm.at[e, pl.ds(HD, HD), :], bufD.at[1], sems.at[5])
        cp.start()

    def wait(e, j):
        if j in (0, 1, 2, 3):
            pltpu.make_async_copy(
                wg_hbm.at[e, pl.ds(0, HF), :], bufA.at[j], sems.at[j]).wait()
        else:
            pltpu.make_async_copy(
                wd_hbm.at[e, pl.ds(0, HD), :], bufD.at[j - 4], sems.at[j]).wait()

    for j in range(6):
        issue(0, j)

    logits = jax.lax.dot_general(
        xf, wr_ref[...], (((1,), (1,)), ((), ())),
        preferred_element_type=jnp.float32)
    m = jnp.max(logits, axis=-1, keepdims=True)
    ex = jnp.exp(logits - m)
    scores = ex / jnp.sum(ex, axis=-1, keepdims=True)
    iota = jax.lax.broadcasted_iota(jnp.int32, scores.shape, 1)
    m1 = jnp.max(scores, axis=-1, keepdims=True)
    i1 = jnp.min(jnp.where(scores == m1, iota, N_EXPERTS),
                 axis=-1, keepdims=True)
    sel1 = iota == i1
    masked = jnp.where(sel1, -jnp.inf, scores)
    m2 = jnp.max(masked, axis=-1, keepdims=True)
    i2 = jnp.min(jnp.where(masked == m2, iota, N_EXPERTS),
                 axis=-1, keepdims=True)
    sel = sel1 | (iota == i2)
    comb = jnp.where(sel, scores, 0.0) / (m1 + m2)
    probs = jnp.mean(scores, axis=0, keepdims=True)
    fracs = jnp.mean(sel.astype(jnp.float32), axis=0, keepdims=True)
    aux_ref[...] = N_EXPERTS * jnp.sum(probs * fracs, keepdims=True)
    acc_ref[...] = jnp.zeros_like(acc_ref)

    def body(e, carry):
        c_e = jnp.sum(jnp.where(iota == e, comb, 0.0),
                      axis=-1, keepdims=True)

        def step(j, compute):
            wait(e, j)
            res = compute()
            @pl.when(e + 1 < N_EXPERTS)
            def _():
                issue(e + 1, j)
            return res

        g0 = step(0, lambda: jax.lax.dot_general(
            x16, bufA[0].astype(jnp.bfloat16), (((1,), (1,)), ((), ())),
            preferred_element_type=jnp.float32))
        u0 = step(1, lambda: jax.lax.dot_general(
            x16, bufA[1].astype(jnp.bfloat16), (((1,), (1,)), ((), ())),
            preferred_element_type=jnp.float32))
        hs0 = ((g0 * jax.lax.logistic(g0)) * u0 * c_e).astype(jnp.bfloat16)
        g1 = step(2, lambda: jax.lax.dot_general(
            x16, bufA[2].astype(jnp.bfloat16), (((1,), (1,)), ((), ())),
            preferred_element_type=jnp.float32))
        u1 = step(3, lambda: jax.lax.dot_general(
            x16, bufA[3].astype(jnp.bfloat16), (((1,), (1,)), ((), ())),
            preferred_element_type=jnp.float32))
        hs1 = ((g1 * jax.lax.logistic(g1)) * u1 * c_e).astype(jnp.bfloat16)
        hs = jnp.concatenate([hs0, hs1], axis=1)
        y0 = step(4, lambda: jax.lax.dot_general(
            hs, bufD[0].astype(jnp.bfloat16), (((1,), (1,)), ((), ())),
            preferred_element_type=jnp.float32))
        acc_ref[:, 0:HD] += y0
        y1 = step(5, lambda: jax.lax.dot_general(
            hs, bufD[1].astype(jnp.bfloat16), (((1,), (1,)), ((), ())),
            preferred_element_type=jnp.float32))
        acc_ref[:, HD:D_MODEL] += y1
        return carry

    jax.lax.fori_loop(0, N_EXPERTS, body, 0)
    out_ref[:, 0, :] = acc_ref[...]


@functools.partial(jax.jit, static_argnums=())
def kernel(x, W_router, W_gate, W_up, W_down):
    Bx, Tx, D = x.shape
    n = Bx * Tx
    out, aux = pl.pallas_call(
        _moe_kernel,
        in_specs=[
            pl.BlockSpec(memory_space=pltpu.MemorySpace.VMEM),
            pl.BlockSpec(memory_space=pltpu.MemorySpace.VMEM),
            pl.BlockSpec(memory_space=pltpu.MemorySpace.HBM),
            pl.BlockSpec(memory_space=pltpu.MemorySpace.HBM),
            pl.BlockSpec(memory_space=pltpu.MemorySpace.HBM),
        ],
        out_specs=[
            pl.BlockSpec(memory_space=pltpu.MemorySpace.VMEM),
            pl.BlockSpec(memory_space=pltpu.MemorySpace.VMEM),
        ],
        out_shape=[
            jax.ShapeDtypeStruct((n, Tx, D), jnp.float32),
            jax.ShapeDtypeStruct((1, 1), jnp.float32),
        ],
        scratch_shapes=[
            pltpu.VMEM((4, HF, D_MODEL), jnp.float32),
            pltpu.VMEM((2, HD, D_FF), jnp.float32),
            pltpu.VMEM((n, D_MODEL), jnp.float32),
            pltpu.SemaphoreType.DMA((6,)),
        ],
    )(x, W_router, W_gate, W_up, W_down)
    return out, aux[0, 0]
